# Initial kernel scaffold; baseline (speedup 1.0000x reference)
import jax
import jax.numpy as jnp
from jax.experimental import pallas as pl


def kernel(x, edge_index, edge_type, fc_W, attn_W, w, w_comp):
    raise NotImplementedError("write your pallas kernel here")


# speedup vs baseline: 21.1600x; 21.1600x over previous
import jax
import jax.numpy as jnp
from jax import lax
from jax.experimental import pallas as pl
from jax.experimental.pallas import tpu as pltpu
from jax.experimental.pallas import tpu_sc as plsc

N = 10000
E = 320000
D = 128
R = 8
SW = 16
NC, NS, L = 2, 16, 16
NW = NC * NS
EPW = E // NW
C = 80
NCH = EPW // C
GPC = C // L
STRIPE = 632
NPAD = NS * STRIPE
NB = 400
NBLK = N // NB


def _dense_body(x_ref, wra_ref, wrb_ref, ca_ref, cb_ref, a2_ref,
                trans_ref, s_ref):
    xb = x_ref[...]
    bt = wra_ref[...] * ca_ref[...] + wrb_ref[...] * cb_ref[...]
    dn = (((1,), (1,)), ((), ()))
    trans_ref[...] = lax.dot_general(xb, bt, dn,
                                     preferred_element_type=jnp.float32)
    s_ref[...] = lax.dot_general(xb, a2_ref[...], (((1,), (0,)), ((), ())),
                                 preferred_element_type=jnp.float32)


_dense_call = pl.pallas_call(
    _dense_body,
    grid=(NBLK,),
    in_specs=[
        pl.BlockSpec((NB, D), lambda i: (i, 0)),
        pl.BlockSpec((R * D, D), lambda i: (0, 0)),
        pl.BlockSpec((R * D, D), lambda i: (0, 0)),
        pl.BlockSpec((R * D, 1), lambda i: (0, 0)),
        pl.BlockSpec((R * D, 1), lambda i: (0, 0)),
        pl.BlockSpec((D, SW), lambda i: (0, 0)),
    ],
    out_specs=[
        pl.BlockSpec((NB, R * D), lambda i: (i, 0)),
        pl.BlockSpec((NB, SW), lambda i: (i, 0)),
    ],
    out_shape=[
        jax.ShapeDtypeStruct((N, R * D), jnp.float32),
        jax.ShapeDtypeStruct((N, SW), jnp.float32),
    ],
)


def _sc_body(trans_h, s_h, src_h, dst_h, et_h, num_o, den_o,
             src_c, dst_c, et_c, row_c, ee_c, s1g, s2g, rows, dstg, sem,
             num_sh, den_sh):
    cid = lax.axis_index("c")
    sid = lax.axis_index("s")
    wid = sid * NC + cid
    ebase = wid * EPW

    zf16 = jnp.zeros((L,), jnp.float32)
    iota16 = lax.iota(jnp.int32, L)
    onehot0 = jnp.where(iota16 == 0, 1.0, 0.0)

    def _zrow(j, carry):
        for b in range(D // L):
            rows[j, pl.ds(b * L, L)] = zf16
        dstg[j, pl.ds(0, L)] = zf16
        return carry

    lax.fori_loop(0, C, _zrow, 0)

    base = sid * STRIPE
    nfull = STRIPE // C
    for k in range(nfull):
        pltpu.sync_copy(rows, num_sh.at[pl.ds(base + k * C, C)])
        pltpu.sync_copy(dstg, den_sh.at[pl.ds(base + k * C, C)])
    rem = STRIPE - nfull * C
    pltpu.sync_copy(rows.at[pl.ds(0, rem)],
                    num_sh.at[pl.ds(base + nfull * C, rem)])
    pltpu.sync_copy(dstg.at[pl.ds(0, rem)],
                    den_sh.at[pl.ds(base + nfull * C, rem)])

    plsc.subcore_barrier()

    def _chunk(c, carry):
        cb = ebase + c * C
        pltpu.sync_copy(src_h.at[pl.ds(cb, C)], src_c)
        pltpu.sync_copy(dst_h.at[pl.ds(cb, C)], dst_c)
        pltpu.sync_copy(et_h.at[pl.ds(cb, C)], et_c)

        def _rid(g, carry2):
            sl = pl.ds(g * L, L)
            row_c[sl] = src_c[sl] * R + et_c[sl]
            return carry2

        lax.fori_loop(0, GPC, _rid, 0)

        cp1 = pltpu.async_copy(s_h.at[src_c], s1g, sem)
        cp2 = pltpu.async_copy(s_h.at[dst_c], s2g, sem)
        cp3 = pltpu.async_copy(trans_h.at[row_c], rows, sem)
        cp1.wait()
        cp2.wait()

        def _ee(g, carry2):
            idx = g * L + iota16
            sv = plsc.load_gather(s1g, [idx, jnp.zeros((L,), jnp.int32)])
            dv = plsc.load_gather(s2g, [idx, jnp.ones((L,), jnp.int32)])
            e = sv + dv
            e = jnp.where(e >= 0.0, e, e * 0.01)
            ee_c[pl.ds(g * L, L)] = jnp.exp(e)
            return carry2

        lax.fori_loop(0, GPC, _ee, 0)
        cp3.wait()

        def _srow(j, carry2):
            ee = plsc.load_gather(ee_c, [jnp.full((L,), j, jnp.int32)])
            for b in range(D // L):
                rows[j, pl.ds(b * L, L)] = rows[j, pl.ds(b * L, L)] * ee
            dstg[j, pl.ds(0, L)] = ee * onehot0
            return carry2

        lax.fori_loop(0, C, _srow, 0)

        pltpu.sync_copy(rows, num_sh.at[dst_c], add=True)
        pltpu.sync_copy(dstg, den_sh.at[dst_c], add=True)
        return carry

    lax.fori_loop(0, NCH, _chunk, 0)

    plsc.subcore_barrier()

    pltpu.sync_copy(num_sh.at[pl.ds(base, STRIPE)],
                    num_o.at[cid, pl.ds(base, STRIPE)])
    pltpu.sync_copy(den_sh.at[pl.ds(base, STRIPE)],
                    den_o.at[cid, pl.ds(base, STRIPE)])


def _make_sc_call():
  mesh = plsc.VectorSubcoreMesh(core_axis_name="c", subcore_axis_name="s",
                                num_cores=NC, num_subcores=NS)
  return pl.kernel(
    _sc_body,
    mesh=mesh,
    compiler_params=pltpu.CompilerParams(needs_layout_passes=False, use_tc_tiling_on_sc=False),
    out_type=[
        jax.ShapeDtypeStruct((NC, NPAD, D), jnp.float32),
        jax.ShapeDtypeStruct((NC, NPAD, L), jnp.float32),
    ],
    scratch_types=[
        pltpu.VMEM((C,), jnp.int32),
        pltpu.VMEM((C,), jnp.int32),
        pltpu.VMEM((C,), jnp.int32),
        pltpu.VMEM((C,), jnp.int32),
        pltpu.VMEM((C,), jnp.float32),
        pltpu.VMEM((C, SW), jnp.float32),
        pltpu.VMEM((C, SW), jnp.float32),
        pltpu.VMEM((C, D), jnp.float32),
        pltpu.VMEM((C, L), jnp.float32),
        pltpu.SemaphoreType.DMA,
        pltpu.VMEM_SHARED((NPAD, D), jnp.float32),
        pltpu.VMEM_SHARED((NPAD, L), jnp.float32),
    ],
  )


def _combine_body(num_ref, den_ref, out_ref):
    nsum = num_ref[0] + num_ref[1]
    d = den_ref[0, :, 0:1] + den_ref[1, :, 0:1]
    out_ref[...] = nsum / jnp.maximum(d, 1e-30)


_combine_call = pl.pallas_call(
    _combine_body,
    grid=(NBLK,),
    in_specs=[
        pl.BlockSpec((NC, NB, D), lambda i: (0, i, 0)),
        pl.BlockSpec((NC, NB, L), lambda i: (0, i, 0)),
    ],
    out_specs=pl.BlockSpec((NB, D), lambda i: (i, 0)),
    out_shape=jax.ShapeDtypeStruct((N, D), jnp.float32),
)


def kernel(x, edge_index, edge_type, fc_W, attn_W, w, w_comp):
    src = edge_index[0]
    dst = edge_index[1]
    et = edge_type[:, 0]

    w3 = w.reshape(D, 2, D)
    wra = jnp.repeat(w3[:, 0, :], R, axis=0)
    wrb = jnp.repeat(w3[:, 1, :], R, axis=0)
    ca = jnp.tile(w_comp[:, 0], D)[:, None]
    cb = jnp.tile(w_comp[:, 1], D)[:, None]
    a2 = jnp.zeros((D, SW), x.dtype).at[:, 0:2].set(attn_W.reshape(2, D).T)

    trans, s_wide = _dense_call(x, wra, wrb, ca, cb, a2)
    trans2 = trans.reshape(N * R, D)

    num, den = _make_sc_call()(trans2, s_wide, src, dst, et)
    return _combine_call(num, den)


# speedup vs baseline: 22.4076x; 1.0590x over previous
import jax
import jax.numpy as jnp
from jax import lax
from jax.experimental import pallas as pl
from jax.experimental.pallas import tpu as pltpu
from jax.experimental.pallas import tpu_sc as plsc

N = 10000
E = 320000
D = 128
R = 8
SW = 16
NC, NS, L = 2, 16, 16
NW = NC * NS
EPW = E // NW
C = 80
NCH = EPW // C
GPC = C // L
STRIPE = 624
TAIL = N - NS * STRIPE
NB = 400
NBLK = N // NB


def _dense_body(x_ref, wra_ref, wrb_ref, ca_ref, cb_ref, a2_ref,
                trans_ref, s_ref):
    xb = x_ref[...]
    bt = wra_ref[...] * ca_ref[...] + wrb_ref[...] * cb_ref[...]
    dn = (((1,), (1,)), ((), ()))
    trans_ref[...] = lax.dot_general(xb, bt, dn,
                                     preferred_element_type=jnp.float32)
    s_ref[...] = lax.dot_general(xb, a2_ref[...], (((1,), (0,)), ((), ())),
                                 preferred_element_type=jnp.float32)


_dense_call = pl.pallas_call(
    _dense_body,
    grid=(NBLK,),
    in_specs=[
        pl.BlockSpec((NB, D), lambda i: (i, 0)),
        pl.BlockSpec((R * D, D), lambda i: (0, 0)),
        pl.BlockSpec((R * D, D), lambda i: (0, 0)),
        pl.BlockSpec((R * D, 1), lambda i: (0, 0)),
        pl.BlockSpec((R * D, 1), lambda i: (0, 0)),
        pl.BlockSpec((D, SW), lambda i: (0, 0)),
    ],
    out_specs=[
        pl.BlockSpec((NB, R * D), lambda i: (i, 0)),
        pl.BlockSpec((NB, SW), lambda i: (i, 0)),
    ],
    out_shape=[
        jax.ShapeDtypeStruct((N, R * D), jnp.float32),
        jax.ShapeDtypeStruct((N, SW), jnp.float32),
    ],
)


def _sc_body(trans_h, s_h, pk_h, num_o, den_o,
             pk0, pk1, row0, row1, dst0, dst1, s1g0, s1g1, s2g0, s2g1,
             rows0, rows1, ee_c, dstg, sp0, sp1, sg0, sg1,
             num_sh, den_sh):
    cid = lax.axis_index("c")
    sid = lax.axis_index("s")
    wid = sid * NC + cid
    cbase = wid * NCH

    zf16 = jnp.zeros((L,), jnp.float32)
    iota16 = lax.iota(jnp.int32, L)
    onehot0 = jnp.where(iota16 == 0, 1.0, 0.0)

    bufs = ((pk0, row0, dst0, s1g0, s2g0, rows0, sp0, sg0),
            (pk1, row1, dst1, s1g1, s2g1, rows1, sp1, sg1))

    def _zrow(j, carry):
        for b in range(D // L):
            rows0[j, pl.ds(b * L, L)] = zf16
        dstg[j, pl.ds(0, L)] = zf16
        return carry

    lax.fori_loop(0, C, _zrow, 0)

    base = sid * STRIPE
    nfull = STRIPE // C
    rem = STRIPE - nfull * C
    for k in range(nfull):
        pltpu.sync_copy(rows0, num_sh.at[pl.ds(base + k * C, C)])
        pltpu.sync_copy(dstg, den_sh.at[pl.ds(base + k * C, C)])
    pltpu.sync_copy(rows0.at[pl.ds(0, rem)],
                    num_sh.at[pl.ds(base + nfull * C, rem)])
    pltpu.sync_copy(dstg.at[pl.ds(0, rem)],
                    den_sh.at[pl.ds(base + nfull * C, rem)])

    @pl.when(sid == 0)
    def _zero_tail():
        pltpu.sync_copy(rows0.at[pl.ds(0, TAIL)],
                        num_sh.at[pl.ds(NS * STRIPE, TAIL)])
        pltpu.sync_copy(dstg.at[pl.ds(0, TAIL)],
                        den_sh.at[pl.ds(NS * STRIPE, TAIL)])

    plsc.subcore_barrier()

    def _issue_pk(c, b):
        pk, _, _, _, _, _, sp, _ = bufs[b]
        return pltpu.async_copy(pk_h.at[cbase + c], pk, sp)

    def _wait_pk(b):
        pk, _, _, _, _, _, sp, _ = bufs[b]
        pltpu.make_async_copy(pk_h.at[0], pk, sp).wait()

    def _prep_and_gather(c, b):
        pk, row_c, dst_c, s1g, s2g, rows, _, sg = bufs[b]
        for g in range(GPC):
            sl = pl.ds(g * L, L)
            row_c[sl] = pk[0, sl] * R + pk[2, sl]
            dst_c[sl] = pk[1, sl]
        pltpu.async_copy(s_h.at[pk.at[0]], s1g, sg)
        pltpu.async_copy(s_h.at[pk.at[1]], s2g, sg)
        pltpu.async_copy(trans_h.at[row_c], rows, sg)

    def _wait_gathers(b):
        pk, row_c, _, s1g, s2g, rows, _, sg = bufs[b]
        pltpu.make_async_copy(s_h.at[pk.at[0]], s1g, sg).wait()
        pltpu.make_async_copy(s_h.at[pk.at[1]], s2g, sg).wait()
        pltpu.make_async_copy(trans_h.at[row_c], rows, sg).wait()

    def _compute_scatter(b):
        _, _, dst_c, s1g, s2g, rows, _, _ = bufs[b]

        def _ee(g, carry):
            idx = g * L + iota16
            sv = plsc.load_gather(s1g, [idx, jnp.zeros((L,), jnp.int32)])
            dv = plsc.load_gather(s2g, [idx, jnp.ones((L,), jnp.int32)])
            e = sv + dv
            e = jnp.where(e >= 0.0, e, e * 0.01)
            ee_c[pl.ds(g * L, L)] = jnp.exp(e)
            return carry

        lax.fori_loop(0, GPC, _ee, 0, unroll=GPC)

        def _srow(j, carry):
            ee = plsc.load_gather(ee_c, [jnp.full((L,), j, jnp.int32)])
            for b2 in range(D // L):
                rows[j, pl.ds(b2 * L, L)] = rows[j, pl.ds(b2 * L, L)] * ee
            dstg[j, pl.ds(0, L)] = ee * onehot0
            return carry

        lax.fori_loop(0, C, _srow, 0, unroll=8)

        pltpu.sync_copy(rows, num_sh.at[dst_c], add=True)
        pltpu.sync_copy(dstg, den_sh.at[dst_c], add=True)

    _issue_pk(0, 0).wait()
    _prep_and_gather(0, 0)
    _issue_pk(1, 1)

    def _pair(p, carry):
        c = p * 2
        for b in (0, 1):
            nb = 1 - b
            _wait_pk(nb)
            _prep_and_gather(c + 1, nb)
            _issue_pk(c + 2, b)
            _wait_gathers(b)
            _compute_scatter(b)
            c = c + 1
        return carry

    lax.fori_loop(0, (NCH - 1) // 2, _pair, 0)

    _wait_pk(1)
    _wait_gathers(0)
    _compute_scatter(0)

    plsc.subcore_barrier()

    pltpu.sync_copy(num_sh.at[pl.ds(base, STRIPE)],
                    num_o.at[cid, pl.ds(base, STRIPE)])
    pltpu.sync_copy(den_sh.at[pl.ds(base, STRIPE)],
                    den_o.at[cid, pl.ds(base, STRIPE)])

    @pl.when(sid == 0)
    def _drain_tail():
        pltpu.sync_copy(num_sh.at[pl.ds(NS * STRIPE, TAIL)],
                        num_o.at[cid, pl.ds(NS * STRIPE, TAIL)])
        pltpu.sync_copy(den_sh.at[pl.ds(NS * STRIPE, TAIL)],
                        den_o.at[cid, pl.ds(NS * STRIPE, TAIL)])


def _make_sc_call():
  mesh = plsc.VectorSubcoreMesh(core_axis_name="c", subcore_axis_name="s",
                                num_cores=NC, num_subcores=NS)
  return pl.kernel(
    _sc_body,
    mesh=mesh,
    compiler_params=pltpu.CompilerParams(needs_layout_passes=False,
                                         use_tc_tiling_on_sc=False),
    out_type=[
        jax.ShapeDtypeStruct((NC, N, D), jnp.float32),
        jax.ShapeDtypeStruct((NC, N, L), jnp.float32),
    ],
    scratch_types=[
        pltpu.VMEM((3, C), jnp.int32),
        pltpu.VMEM((3, C), jnp.int32),
        pltpu.VMEM((C,), jnp.int32),
        pltpu.VMEM((C,), jnp.int32),
        pltpu.VMEM((C,), jnp.int32),
        pltpu.VMEM((C,), jnp.int32),
        pltpu.VMEM((C, SW), jnp.float32),
        pltpu.VMEM((C, SW), jnp.float32),
        pltpu.VMEM((C, SW), jnp.float32),
        pltpu.VMEM((C, SW), jnp.float32),
        pltpu.VMEM((C, D), jnp.float32),
        pltpu.VMEM((C, D), jnp.float32),
        pltpu.VMEM((C,), jnp.float32),
        pltpu.VMEM((C, L), jnp.float32),
        pltpu.SemaphoreType.DMA,
        pltpu.SemaphoreType.DMA,
        pltpu.SemaphoreType.DMA,
        pltpu.SemaphoreType.DMA,
        pltpu.VMEM_SHARED((N, D), jnp.float32),
        pltpu.VMEM_SHARED((N, L), jnp.float32),
    ],
  )


def _combine_body(num_ref, den_ref, out_ref):
    nsum = num_ref[0] + num_ref[1]
    d = den_ref[0, :, 0:1] + den_ref[1, :, 0:1]
    out_ref[...] = nsum / jnp.maximum(d, 1e-30)


_combine_call = pl.pallas_call(
    _combine_body,
    grid=(NBLK,),
    in_specs=[
        pl.BlockSpec((NC, NB, D), lambda i: (0, i, 0)),
        pl.BlockSpec((NC, NB, L), lambda i: (0, i, 0)),
    ],
    out_specs=pl.BlockSpec((NB, D), lambda i: (i, 0)),
    out_shape=jax.ShapeDtypeStruct((N, D), jnp.float32),
)


def kernel(x, edge_index, edge_type, fc_W, attn_W, w, w_comp):
    src = edge_index[0]
    dst = edge_index[1]
    et = edge_type[:, 0]

    w3 = w.reshape(D, 2, D)
    wra = jnp.repeat(w3[:, 0, :], R, axis=0)
    wrb = jnp.repeat(w3[:, 1, :], R, axis=0)
    ca = jnp.tile(w_comp[:, 0], D)[:, None]
    cb = jnp.tile(w_comp[:, 1], D)[:, None]
    a2 = jnp.zeros((D, SW), x.dtype).at[:, 0:2].set(attn_W.reshape(2, D).T)

    trans, s_wide = _dense_call(x, wra, wrb, ca, cb, a2)
    trans2 = trans.reshape(N * R, D)

    pk = jnp.stack([a.reshape(NW * NCH, C) for a in (src, dst, et)], axis=1)
    pk = jnp.concatenate([pk, jnp.zeros((1, 3, C), jnp.int32)], axis=0)

    num, den = _make_sc_call()(trans2, s_wide, pk)
    return _combine_call(num, den)


# speedup vs baseline: 36.3869x; 1.6239x over previous
import jax
import jax.numpy as jnp
import numpy as np
from jax import lax
from jax.experimental import pallas as pl
from jax.experimental.pallas import tpu as pltpu
from jax.experimental.pallas import tpu_sc as plsc

N = 10000
E = 320000
D = 128
R = 8
SW = 16
NC, NS, L = 2, 16, 16
NW = NC * NS
EPW = E // NW
C = 80
NCH = EPW // C
GPC = C // L
STRIPE = 624
TAIL = N - NS * STRIPE
NB = 400
NBLK = N // NB

import numpy as _np
_PERM = _np.empty(R * D, _np.int32)
for _r in range(R):
    for _j in range(D // 2):
        _PERM[_r * D + 2 * _j] = _r * D + _j
        _PERM[_r * D + 2 * _j + 1] = _r * D + D // 2 + _j


def _dense_body(x_ref, wra_ref, wrb_ref, ca_ref, cb_ref, a2_ref,
                trans_ref, s_ref):
    xb = x_ref[...]
    bt = wra_ref[...] * ca_ref[...] + wrb_ref[...] * cb_ref[...]
    dn = (((1,), (1,)), ((), ()))
    trans_ref[...] = lax.dot_general(
        xb, bt, dn, preferred_element_type=jnp.float32).astype(jnp.bfloat16)
    s_ref[...] = lax.dot_general(xb, a2_ref[...], (((1,), (0,)), ((), ())),
                                 preferred_element_type=jnp.float32)


_dense_call = pl.pallas_call(
    _dense_body,
    grid=(NBLK,),
    in_specs=[
        pl.BlockSpec((NB, D), lambda i: (i, 0)),
        pl.BlockSpec((R * D, D), lambda i: (0, 0)),
        pl.BlockSpec((R * D, D), lambda i: (0, 0)),
        pl.BlockSpec((R * D, 1), lambda i: (0, 0)),
        pl.BlockSpec((R * D, 1), lambda i: (0, 0)),
        pl.BlockSpec((D, SW), lambda i: (0, 0)),
    ],
    out_specs=[
        pl.BlockSpec((NB, R * D), lambda i: (i, 0)),
        pl.BlockSpec((NB, SW), lambda i: (i, 0)),
    ],
    out_shape=[
        jax.ShapeDtypeStruct((N, R * D), jnp.bfloat16),
        jax.ShapeDtypeStruct((N, SW), jnp.float32),
    ],
)


def _sc_body(trans_h, s_h, pk_h, num_o, den_o,
             pk0, pk1, row0, row1, dst0, dst1, s1g0, s1g1, s2g0, s2g1,
             rows0, rows1, rowsf, ee_c, dstg, sp0, sp1, sg0, sg1,
             num_sh, den_sh):
    cid = lax.axis_index("c")
    sid = lax.axis_index("s")
    wid = sid * NC + cid
    cbase = wid * NCH

    zf16 = jnp.zeros((L,), jnp.float32)
    iota16 = lax.iota(jnp.int32, L)
    onehot0 = jnp.where(iota16 == 0, 1.0, 0.0)

    bufs = ((pk0, row0, dst0, s1g0, s2g0, rows0, sp0, sg0),
            (pk1, row1, dst1, s1g1, s2g1, rows1, sp1, sg1))

    def _zrow(j, carry):
        for b in range(D // L):
            rowsf[j, pl.ds(b * L, L)] = zf16
        dstg[j, pl.ds(0, L)] = zf16
        return carry

    lax.fori_loop(0, C, _zrow, 0)

    base = sid * STRIPE
    nfull = STRIPE // C
    rem = STRIPE - nfull * C
    for k in range(nfull):
        pltpu.sync_copy(rowsf, num_sh.at[pl.ds(base + k * C, C)])
        pltpu.sync_copy(dstg, den_sh.at[pl.ds(base + k * C, C)])
    pltpu.sync_copy(rowsf.at[pl.ds(0, rem)],
                    num_sh.at[pl.ds(base + nfull * C, rem)])
    pltpu.sync_copy(dstg.at[pl.ds(0, rem)],
                    den_sh.at[pl.ds(base + nfull * C, rem)])

    @pl.when(sid == 0)
    def _zero_tail():
        pltpu.sync_copy(rowsf.at[pl.ds(0, TAIL)],
                        num_sh.at[pl.ds(NS * STRIPE, TAIL)])
        pltpu.sync_copy(dstg.at[pl.ds(0, TAIL)],
                        den_sh.at[pl.ds(NS * STRIPE, TAIL)])

    plsc.subcore_barrier()

    def _issue_pk(c, b):
        pk, _, _, _, _, _, sp, _ = bufs[b]
        return pltpu.async_copy(pk_h.at[cbase + c], pk, sp)

    def _wait_pk(b):
        pk, _, _, _, _, _, sp, _ = bufs[b]
        pltpu.make_async_copy(pk_h.at[0], pk, sp).wait()

    def _prep_and_gather(c, b):
        pk, row_c, dst_c, s1g, s2g, rows, _, sg = bufs[b]
        for g in range(GPC):
            sl = pl.ds(g * L, L)
            row_c[sl] = pk[0, sl] * R + pk[2, sl]
            dst_c[sl] = pk[1, sl]
        pltpu.async_copy(trans_h.at[row_c], rows, sg)
        pltpu.async_copy(s_h.at[pk.at[0]], s1g, sg)
        pltpu.async_copy(s_h.at[pk.at[1]], s2g, sg)

    def _wait_gathers(b):
        pk, row_c, _, s1g, s2g, rows, _, sg = bufs[b]
        pltpu.make_async_copy(trans_h.at[row_c], rows, sg).wait()
        pltpu.make_async_copy(s_h.at[pk.at[0]], s1g, sg).wait()
        pltpu.make_async_copy(s_h.at[pk.at[1]], s2g, sg).wait()

    def _compute_scatter(b):
        _, _, dst_c, s1g, s2g, rows, _, _ = bufs[b]

        @plsc.parallel_loop(0, GPC, unroll=GPC)
        def _ee(g):
            idx = g * L + iota16
            sv = plsc.load_gather(s1g, [idx, jnp.zeros((L,), jnp.int32)])
            dv = plsc.load_gather(s2g, [idx, jnp.ones((L,), jnp.int32)])
            e = sv + dv
            e = jnp.where(e >= 0.0, e, e * 0.01)
            ee_c[pl.ds(g * L, L)] = jnp.exp(e)

        @plsc.parallel_loop(0, C, unroll=8)
        def _srow(j):
            ee = plsc.load_gather(ee_c, [jnp.full((L,), j, jnp.int32)])
            for g in range(D // (2 * L)):
                vi = plsc.bitcast(rows[j, pl.ds(g * 2 * L, 2 * L)], jnp.int32)
                fe = plsc.bitcast(vi << 16, jnp.float32)
                fo = plsc.bitcast(vi & jnp.int32(-65536), jnp.float32)
                rowsf[j, pl.ds(g * L, L)] = fe * ee
                rowsf[j, pl.ds(D // 2 + g * L, L)] = fo * ee
            dstg[j, pl.ds(0, L)] = ee * onehot0

        pltpu.sync_copy(rowsf, num_sh.at[dst_c], add=True)
        pltpu.sync_copy(dstg, den_sh.at[dst_c], add=True)

    _issue_pk(0, 0).wait()
    _prep_and_gather(0, 0)
    _issue_pk(1, 1)

    def _pair(p, carry):
        c = p * 2
        for b in (0, 1):
            nb = 1 - b
            _wait_pk(nb)
            _prep_and_gather(c + 1, nb)
            _issue_pk(c + 2, b)
            _wait_gathers(b)
            _compute_scatter(b)
            c = c + 1
        return carry

    lax.fori_loop(0, (NCH - 1) // 2, _pair, 0)

    _wait_pk(1)
    _wait_gathers(0)
    _compute_scatter(0)

    plsc.subcore_barrier()

    pltpu.sync_copy(num_sh.at[pl.ds(base, STRIPE)],
                    num_o.at[cid, pl.ds(base, STRIPE)])
    pltpu.sync_copy(den_sh.at[pl.ds(base, STRIPE)],
                    den_o.at[cid, pl.ds(base, STRIPE)])

    @pl.when(sid == 0)
    def _drain_tail():
        pltpu.sync_copy(num_sh.at[pl.ds(NS * STRIPE, TAIL)],
                        num_o.at[cid, pl.ds(NS * STRIPE, TAIL)])
        pltpu.sync_copy(den_sh.at[pl.ds(NS * STRIPE, TAIL)],
                        den_o.at[cid, pl.ds(NS * STRIPE, TAIL)])


def _make_sc_call():
  mesh = plsc.VectorSubcoreMesh(core_axis_name="c", subcore_axis_name="s",
                                num_cores=NC, num_subcores=NS)
  return pl.kernel(
    _sc_body,
    mesh=mesh,
    compiler_params=pltpu.CompilerParams(needs_layout_passes=False,
                                         use_tc_tiling_on_sc=False),
    out_type=[
        jax.ShapeDtypeStruct((NC, N, D), jnp.float32),
        jax.ShapeDtypeStruct((NC, N, L), jnp.float32),
    ],
    scratch_types=[
        pltpu.VMEM((3, C), jnp.int32),
        pltpu.VMEM((3, C), jnp.int32),
        pltpu.VMEM((C,), jnp.int32),
        pltpu.VMEM((C,), jnp.int32),
        pltpu.VMEM((C,), jnp.int32),
        pltpu.VMEM((C,), jnp.int32),
        pltpu.VMEM((C, SW), jnp.float32),
        pltpu.VMEM((C, SW), jnp.float32),
        pltpu.VMEM((C, SW), jnp.float32),
        pltpu.VMEM((C, SW), jnp.float32),
        pltpu.VMEM((C, D), jnp.bfloat16),
        pltpu.VMEM((C, D), jnp.bfloat16),
        pltpu.VMEM((C, D), jnp.float32),
        pltpu.VMEM((C,), jnp.float32),
        pltpu.VMEM((C, L), jnp.float32),
        pltpu.SemaphoreType.DMA,
        pltpu.SemaphoreType.DMA,
        pltpu.SemaphoreType.DMA,
        pltpu.SemaphoreType.DMA,
        pltpu.VMEM_SHARED((N, D), jnp.float32),
        pltpu.VMEM_SHARED((N, L), jnp.float32),
    ],
  )


def _combine_body(num_ref, den_ref, out_ref):
    nsum = num_ref[0] + num_ref[1]
    d = den_ref[0, :, 0:1] + den_ref[1, :, 0:1]
    out_ref[...] = nsum / jnp.maximum(d, 1e-30)


_combine_call = pl.pallas_call(
    _combine_body,
    grid=(NBLK,),
    in_specs=[
        pl.BlockSpec((NC, NB, D), lambda i: (0, i, 0)),
        pl.BlockSpec((NC, NB, L), lambda i: (0, i, 0)),
    ],
    out_specs=pl.BlockSpec((NB, D), lambda i: (i, 0)),
    out_shape=jax.ShapeDtypeStruct((N, D), jnp.float32),
)


def kernel(x, edge_index, edge_type, fc_W, attn_W, w, w_comp):
    src = edge_index[0]
    dst = edge_index[1]
    et = edge_type[:, 0]

    w3 = w.reshape(D, 2, D)
    wra = jnp.repeat(w3[:, 0, :], R, axis=0)[_PERM]
    wrb = jnp.repeat(w3[:, 1, :], R, axis=0)[_PERM]
    ca = jnp.tile(w_comp[:, 0], D)[_PERM][:, None]
    cb = jnp.tile(w_comp[:, 1], D)[_PERM][:, None]
    a2 = jnp.zeros((D, SW), x.dtype).at[:, 0:2].set(attn_W.reshape(2, D).T)

    trans, s_wide = _dense_call(x, wra, wrb, ca, cb, a2)
    trans2 = trans.reshape(N * R, D)

    pk = jnp.stack([a.reshape(NW * NCH, C) for a in (src, dst, et)], axis=1)
    pk = jnp.concatenate([pk, jnp.zeros((1, 3, C), jnp.int32)], axis=0)

    num, den = _make_sc_call()(trans2, s_wide, pk)
    return _combine_call(num, den)


# speedup vs baseline: 43.0152x; 1.1822x over previous
import jax
import jax.numpy as jnp
from jax import lax
from jax.experimental import pallas as pl
from jax.experimental.pallas import tpu as pltpu
from jax.experimental.pallas import tpu_sc as plsc

N = 10000
E = 320000
D = 128
R = 8
SW = 16
NC, NS, L = 2, 16, 16
NW = NC * NS
EPW = E // NW
C = 80
NCH = EPW // C
GPC = C // L
STRIPE = 624
TAIL = N - NS * STRIPE
NB = 400
NBLK = N // NB


def _dense_body(x_ref, wra_ref, wrb_ref, ca_ref, cb_ref, a2_ref,
                trans_ref, s_ref):
    xb = x_ref[...]
    bt = wra_ref[...] * ca_ref[...] + wrb_ref[...] * cb_ref[...]
    dn = (((1,), (1,)), ((), ()))
    trans_ref[...] = lax.dot_general(xb, bt, dn,
                                     preferred_element_type=jnp.float32)
    s_ref[...] = lax.dot_general(xb, a2_ref[...], (((1,), (0,)), ((), ())),
                                 preferred_element_type=jnp.float32)


_dense_call = pl.pallas_call(
    _dense_body,
    grid=(NBLK,),
    in_specs=[
        pl.BlockSpec((NB, D), lambda i: (i, 0)),
        pl.BlockSpec((R * D, D), lambda i: (0, 0)),
        pl.BlockSpec((R * D, D), lambda i: (0, 0)),
        pl.BlockSpec((R * D, 1), lambda i: (0, 0)),
        pl.BlockSpec((R * D, 1), lambda i: (0, 0)),
        pl.BlockSpec((D, SW), lambda i: (0, 0)),
    ],
    out_specs=[
        pl.BlockSpec((NB, R * D), lambda i: (i, 0)),
        pl.BlockSpec((NB, SW), lambda i: (i, 0)),
    ],
    out_shape=[
        jax.ShapeDtypeStruct((N, R * D), jnp.float32),
        jax.ShapeDtypeStruct((N, SW), jnp.float32),
    ],
)


def _sc_body(trans_h, s_h, pk_h, num_o, den_o,
             pk0, pk1, row0, row1, dst0, dst1, s1g0, s1g1, s2g0, s2g1,
             rows0, rows1, ee_c, dstg, sp0, sp1, sg0, sg1,
             num_sh, den_sh):
    cid = lax.axis_index("c")
    sid = lax.axis_index("s")
    wid = sid * NC + cid
    cbase = wid * NCH

    zf16 = jnp.zeros((L,), jnp.float32)
    iota16 = lax.iota(jnp.int32, L)
    onehot0 = jnp.where(iota16 == 0, 1.0, 0.0)

    bufs = ((pk0, row0, dst0, s1g0, s2g0, rows0, sp0, sg0),
            (pk1, row1, dst1, s1g1, s2g1, rows1, sp1, sg1))

    def _zrow(j, carry):
        for b in range(D // L):
            rows0[j, pl.ds(b * L, L)] = zf16
        dstg[j, pl.ds(0, L)] = zf16
        return carry

    lax.fori_loop(0, C, _zrow, 0)

    base = sid * STRIPE
    nfull = STRIPE // C
    rem = STRIPE - nfull * C
    for k in range(nfull):
        pltpu.sync_copy(rows0, num_sh.at[pl.ds(base + k * C, C)])
        pltpu.sync_copy(dstg, den_sh.at[pl.ds(base + k * C, C)])
    pltpu.sync_copy(rows0.at[pl.ds(0, rem)],
                    num_sh.at[pl.ds(base + nfull * C, rem)])
    pltpu.sync_copy(dstg.at[pl.ds(0, rem)],
                    den_sh.at[pl.ds(base + nfull * C, rem)])

    @pl.when(sid == 0)
    def _zero_tail():
        pltpu.sync_copy(rows0.at[pl.ds(0, TAIL)],
                        num_sh.at[pl.ds(NS * STRIPE, TAIL)])
        pltpu.sync_copy(dstg.at[pl.ds(0, TAIL)],
                        den_sh.at[pl.ds(NS * STRIPE, TAIL)])

    plsc.subcore_barrier()

    def _issue_pk(c, b):
        pk, _, _, _, _, _, sp, _ = bufs[b]
        return pltpu.async_copy(pk_h.at[cbase + c], pk, sp)

    def _wait_pk(b):
        pk, _, _, _, _, _, sp, _ = bufs[b]
        pltpu.make_async_copy(pk_h.at[0], pk, sp).wait()

    def _prep_and_gather(c, b):
        pk, row_c, dst_c, s1g, s2g, rows, _, sg = bufs[b]
        for g in range(GPC):
            sl = pl.ds(g * L, L)
            row_c[sl] = pk[0, sl] * R + pk[2, sl]
            dst_c[sl] = pk[1, sl]
        pltpu.async_copy(trans_h.at[row_c], rows, sg)
        pltpu.async_copy(s_h.at[pk.at[0]], s1g, sg)
        pltpu.async_copy(s_h.at[pk.at[1]], s2g, sg)

    def _wait_gathers(b):
        pk, row_c, _, s1g, s2g, rows, _, sg = bufs[b]
        pltpu.make_async_copy(trans_h.at[row_c], rows, sg).wait()
        pltpu.make_async_copy(s_h.at[pk.at[0]], s1g, sg).wait()
        pltpu.make_async_copy(s_h.at[pk.at[1]], s2g, sg).wait()

    def _compute_scatter(b):
        _, _, dst_c, s1g, s2g, rows, _, _ = bufs[b]

        @plsc.parallel_loop(0, GPC, unroll=GPC)
        def _ee(g):
            idx = g * L + iota16
            sv = plsc.load_gather(s1g, [idx, jnp.zeros((L,), jnp.int32)])
            dv = plsc.load_gather(s2g, [idx, jnp.ones((L,), jnp.int32)])
            e = sv + dv
            e = jnp.where(e >= 0.0, e, e * 0.01)
            ee_c[pl.ds(g * L, L)] = jnp.exp(e)

        @plsc.parallel_loop(0, C, unroll=8)
        def _srow(j):
            ee = plsc.load_gather(ee_c, [jnp.full((L,), j, jnp.int32)])
            for b2 in range(D // L):
                rows[j, pl.ds(b2 * L, L)] = rows[j, pl.ds(b2 * L, L)] * ee
            dstg[j, pl.ds(0, L)] = ee * onehot0

        pltpu.sync_copy(rows, num_sh.at[dst_c], add=True)
        pltpu.sync_copy(dstg, den_sh.at[dst_c], add=True)

    _issue_pk(0, 0).wait()
    _prep_and_gather(0, 0)
    _issue_pk(1, 1)

    def _pair(p, carry):
        c = p * 2
        for b in (0, 1):
            nb = 1 - b
            _wait_pk(nb)
            _prep_and_gather(c + 1, nb)
            _issue_pk(c + 2, b)
            _wait_gathers(b)
            _compute_scatter(b)
            c = c + 1
        return carry

    lax.fori_loop(0, (NCH - 1) // 2, _pair, 0)

    _wait_pk(1)
    _wait_gathers(0)
    _compute_scatter(0)

    plsc.subcore_barrier()

    pltpu.sync_copy(num_sh.at[pl.ds(base, STRIPE)],
                    num_o.at[cid, pl.ds(base, STRIPE)])
    pltpu.sync_copy(den_sh.at[pl.ds(base, STRIPE)],
                    den_o.at[cid, pl.ds(base, STRIPE)])

    @pl.when(sid == 0)
    def _drain_tail():
        pltpu.sync_copy(num_sh.at[pl.ds(NS * STRIPE, TAIL)],
                        num_o.at[cid, pl.ds(NS * STRIPE, TAIL)])
        pltpu.sync_copy(den_sh.at[pl.ds(NS * STRIPE, TAIL)],
                        den_o.at[cid, pl.ds(NS * STRIPE, TAIL)])


def _make_sc_call():
  mesh = plsc.VectorSubcoreMesh(core_axis_name="c", subcore_axis_name="s",
                                num_cores=NC, num_subcores=NS)
  return pl.kernel(
    _sc_body,
    mesh=mesh,
    compiler_params=pltpu.CompilerParams(needs_layout_passes=False,
                                         use_tc_tiling_on_sc=False),
    out_type=[
        jax.ShapeDtypeStruct((NC, N, D), jnp.float32),
        jax.ShapeDtypeStruct((NC, N, L), jnp.float32),
    ],
    scratch_types=[
        pltpu.VMEM((3, C), jnp.int32),
        pltpu.VMEM((3, C), jnp.int32),
        pltpu.VMEM((C,), jnp.int32),
        pltpu.VMEM((C,), jnp.int32),
        pltpu.VMEM((C,), jnp.int32),
        pltpu.VMEM((C,), jnp.int32),
        pltpu.VMEM((C, SW), jnp.float32),
        pltpu.VMEM((C, SW), jnp.float32),
        pltpu.VMEM((C, SW), jnp.float32),
        pltpu.VMEM((C, SW), jnp.float32),
        pltpu.VMEM((C, D), jnp.float32),
        pltpu.VMEM((C, D), jnp.float32),
        pltpu.VMEM((C,), jnp.float32),
        pltpu.VMEM((C, L), jnp.float32),
        pltpu.SemaphoreType.DMA,
        pltpu.SemaphoreType.DMA,
        pltpu.SemaphoreType.DMA,
        pltpu.SemaphoreType.DMA,
        pltpu.VMEM_SHARED((N, D), jnp.float32),
        pltpu.VMEM_SHARED((N, L), jnp.float32),
    ],
  )


def _combine_body(num_ref, den_ref, out_ref):
    nsum = num_ref[0] + num_ref[1]
    d = den_ref[0, :, 0:1] + den_ref[1, :, 0:1]
    out_ref[...] = nsum / jnp.maximum(d, 1e-30)


_combine_call = pl.pallas_call(
    _combine_body,
    grid=(NBLK,),
    in_specs=[
        pl.BlockSpec((NC, NB, D), lambda i: (0, i, 0)),
        pl.BlockSpec((NC, NB, L), lambda i: (0, i, 0)),
    ],
    out_specs=pl.BlockSpec((NB, D), lambda i: (i, 0)),
    out_shape=jax.ShapeDtypeStruct((N, D), jnp.float32),
)


def kernel(x, edge_index, edge_type, fc_W, attn_W, w, w_comp):
    src = edge_index[0]
    dst = edge_index[1]
    et = edge_type[:, 0]

    w3 = w.reshape(D, 2, D)
    wra = jnp.repeat(w3[:, 0, :], R, axis=0)
    wrb = jnp.repeat(w3[:, 1, :], R, axis=0)
    ca = jnp.tile(w_comp[:, 0], D)[:, None]
    cb = jnp.tile(w_comp[:, 1], D)[:, None]
    a2 = jnp.zeros((D, SW), x.dtype).at[:, 0:2].set(attn_W.reshape(2, D).T)

    trans, s_wide = _dense_call(x, wra, wrb, ca, cb, a2)
    trans2 = trans.reshape(N * R, D)

    pk = jnp.stack([a.reshape(NW * NCH, C) for a in (src, dst, et)], axis=1)
    pk = jnp.concatenate([pk, jnp.zeros((1, 3, C), jnp.int32)], axis=0)

    num, den = _make_sc_call()(trans2, s_wide, pk)
    return _combine_call(num, den)
